# Initial kernel scaffold; baseline (speedup 1.0000x reference)
import jax
import jax.numpy as jnp
from jax.experimental import pallas as pl


def kernel(features, adjacency, mask, We1, be1, We2, be2, Wg, bg, Wgd, bgd, Wp1, bp1, Wp2, bp2, Wv, bv, Wmu, bmu, WL, bL):
    raise NotImplementedError("write your pallas kernel here")


# speedup vs baseline: 3550.1324x; 3550.1324x over previous
import jax
import jax.numpy as jnp
import numpy as np
from jax.experimental import pallas as pl


def _body(featT, adj, maskr, epsT, W1T, b1, W2T, b2, WgT, bgc, WgdT, bgd,
          Wp1aT, Wp1bT, bp1, Wp2T, bp2, WhT, bh, actT_o, valT_o):
    f32 = jnp.float32
    X1 = jax.nn.relu(jnp.dot(W1T[:], featT[:], preferred_element_type=f32) + b1[:])
    XT = jax.nn.relu(jnp.dot(W2T[:], X1, preferred_element_type=f32) + b2[:])
    XlT = jnp.dot(WgT[:], XT, preferred_element_type=f32)
    A = adj[:]
    deg = jnp.sum(A, axis=0, keepdims=True) + 1.0
    dinv = jnp.where(deg > 0, 1.0 / jnp.sqrt(deg), 0.0)
    ST = XlT * dinv
    Y0T = jnp.dot(ST, A, preferred_element_type=f32)
    YT = Y0T * dinv + XlT * (dinv * dinv)
    XgT = jax.nn.relu(YT + bgc[:])
    Xg2T = jax.nn.relu(jnp.dot(WgdT[:], XgT, preferred_element_type=f32) + bgd[:])
    XpT = jax.nn.relu(jnp.dot(Wp1aT[:], Xg2T, preferred_element_type=f32)
                      + jnp.dot(Wp1bT[:], XT, preferred_element_type=f32) + bp1[:])
    XpT = jax.nn.relu(jnp.dot(Wp2T[:], XpT, preferred_element_type=f32) + bp2[:])
    HT = jnp.dot(WhT[:], XpT, preferred_element_type=f32) + bh[:]
    valT_o[:] = HT[0:1, :]
    muT = jnp.tanh(HT[1:3, :])
    zT = jnp.tanh(HT[3:6, :])
    sigT = jnp.concatenate([jnp.exp(-zT[0:1, :]), jnp.exp(-zT[2:3, :])], axis=0)
    act = jnp.clip(muT + epsT[:] * sigT, -1.0, 1.0) * maskr[:]
    actT_o[:] = act


def kernel(features, adjacency, mask, We1, be1, We2, be2, Wg, bg, Wgd, bgd,
           Wp1, bp1, Wp2, bp2, Wv, bv, Wmu, bmu, WL, bL):
    n = features.shape[0]
    A = Wmu.shape[1]
    eps = jax.random.normal(jax.random.key(42), (n, A), jnp.float32)
    Wh = jnp.concatenate([Wv, Wmu, WL], axis=1)
    bh = jnp.concatenate([bv, bmu, bL], axis=0)
    args = (
        features.T,
        adjacency,
        mask.reshape(1, n),
        eps.T,
        We1.T, be1.reshape(-1, 1),
        We2.T, be2.reshape(-1, 1),
        Wg.T, bg.reshape(-1, 1),
        Wgd.T, bgd.reshape(-1, 1),
        Wp1[:32].T, Wp1[32:].T, bp1.reshape(-1, 1),
        Wp2.T, bp2.reshape(-1, 1),
        Wh.T, bh.reshape(-1, 1),
    )
    actT, valT = pl.pallas_call(
        _body,
        out_shape=(
            jax.ShapeDtypeStruct((A, n), jnp.float32),
            jax.ShapeDtypeStruct((1, n), jnp.float32),
        ),
    )(*args)
    return (actT.T, valT.T)
